# baseline (speedup 1.0000x reference)
import functools

import jax
import jax.numpy as jnp
from jax import lax
from jax.experimental import pallas as pl
from jax.experimental.pallas import tpu as pltpu
from jax.experimental.pallas import tpu_sc as plsc

_NC = 2
_NS = 16
_L = 16
_B = 80

_LN2 = 0.6931471805599453


def _sc_sqrt(m):
    i = plsc.bitcast(m, jnp.int32)
    r = plsc.bitcast(
        jnp.int32(0x5F3759DF) - lax.shift_right_logical(i, 1), jnp.float32)
    r = r * (1.5 - 0.5 * m * r * r)
    r = r * (1.5 - 0.5 * m * r * r)
    return m * r


def _sc_log(u):
    i = plsc.bitcast(u, jnp.int32)
    ee = lax.shift_right_logical(i, 23) - 127
    mant = plsc.bitcast(
        jnp.bitwise_or(jnp.bitwise_and(i, jnp.int32(0x007FFFFF)),
                       jnp.int32(0x3F800000)), jnp.float32)
    big = mant > jnp.float32(1.4142135)
    mant = jnp.where(big, mant * 0.5, mant)
    ef = (ee + jnp.where(big, 1, 0)).astype(jnp.float32)
    t = (mant - 1.0) / (mant + 1.0)
    t2 = t * t
    logm = 2.0 * t * (1.0 + t2 * (1.0 / 3.0 + t2 * (1.0 / 5.0)))
    return ef * jnp.float32(_LN2) + logm


def _sc_tanh(h):
    h2 = h * h
    small = h * (1.0 + h2 * (-1.0 / 3.0 + h2 * (2.0 / 15.0
                                                + h2 * (-17.0 / 315.0))))
    big = 1.0 - 2.0 / (jnp.exp(2.0 * h) + 1.0)
    return jnp.where(h > 0.25, big, small)


def _bf16_pair_mulsum(wa, wb):
    va = plsc.bitcast(wa, jnp.bfloat16)
    vb = plsc.bitcast(wb, jnp.bfloat16)
    w = plsc.bitcast(va * vb, jnp.int32)
    hi = plsc.bitcast(jnp.bitwise_and(w, jnp.int32(-65536)), jnp.float32)
    lo = plsc.bitcast(lax.shift_left(w, jnp.int32(16)), jnp.float32)
    return hi + lo


def _tree(parts):
    parts = list(parts)
    while len(parts) > 1:
        parts = [parts[i] + parts[i + 1]
                 for i in range(0, len(parts) - 1, 2)] + (
                     [parts[-1]] if len(parts) % 2 else [])
    return parts[0]


def _fused(xp, src, dst, beta, con):
    npad, dp = xp.shape
    e = src.shape[0]
    nw = _NC * _NS
    epw = e // nw
    nblk = epw // _B
    ngrp = _B // _L
    nchw = dp // _L
    accrows = npad // 128
    rows_per_tile = npad // _NS
    sqchunks = rows_per_tile // _B
    assert epw * nw == e and nblk * _B == epw and nblk % 2 == 1
    assert accrows <= 128 and sqchunks * _B == rows_per_tile
    mesh = plsc.VectorSubcoreMesh(core_axis_name="c", subcore_axis_name="s")

    @functools.partial(
        pl.kernel,
        out_type=[
            jax.ShapeDtypeStruct((e,), jnp.float32),
            jax.ShapeDtypeStruct((_NC, accrows, 128), jnp.float32),
        ],
        mesh=mesh,
        compiler_params=pltpu.CompilerParams(needs_layout_passes=False,
                                             use_tc_tiling_on_sc=False),
        scratch_types=[
            pltpu.VMEM((npad,), jnp.float32),
            pltpu.VMEM((epw,), jnp.int32),
            pltpu.VMEM((epw,), jnp.int32),
            pltpu.VMEM((epw,), jnp.float32),
            pltpu.VMEM((_B, dp), jnp.int32),
            pltpu.VMEM((_B, dp), jnp.int32),
            pltpu.VMEM((_B, dp), jnp.int32),
            pltpu.VMEM((_B, dp), jnp.int32),
            pltpu.VMEM((_L, _L + 1), jnp.float32),
            pltpu.VMEM((accrows, 128), jnp.float32),
            pltpu.VMEM((accrows,), jnp.int32),
            pltpu.VMEM((_L,), jnp.float32),
            pltpu.VMEM((_L,), jnp.float32),
            pltpu.VMEM_SHARED((npad,), jnp.float32),
            pltpu.VMEM_SHARED((accrows, 128), jnp.float32),
            pltpu.SemaphoreType.DMA,
            pltpu.SemaphoreType.DMA,
            pltpu.SemaphoreType.DMA,
            pltpu.SemaphoreType.DMA,
        ],
    )
    def k(xp_hbm, src_hbm, dst_hbm, beta_hbm, con_hbm, ee_hbm, part_hbm,
          sqtab, idx_s, idx_d, ee_all, rs0, rs1, rd0, rd1, tbuf, acc, rowid,
          betav, conv, sh_sq, sh_acc, ss0, ss1, sd0, sd1):
        c = lax.axis_index("c")
        s = lax.axis_index("s")
        wid = s * _NC + c
        tbase = wid * epw
        lane = lax.iota(jnp.int32, _L)
        zz = jnp.zeros((_L,), jnp.float32)

        pltpu.sync_copy(beta_hbm, betav)
        pltpu.sync_copy(con_hbm, conv)

        sqbase = s * rows_per_tile

        @pl.loop(0, sqchunks)
        def _sqc(ch):
            pltpu.sync_copy(xp_hbm.at[pl.ds(sqbase + ch * _B, _B)], rs0)

            @pl.loop(0, ngrp)
            def _sqg(g):
                for ee_ in range(_L):
                    row = g * _L + ee_
                    plsc.store_scatter(
                        tbuf, [lane, jnp.full((_L,), ee_, jnp.int32)],
                        _tree([_bf16_pair_mulsum(rs0[row, pl.ds(cc * _L, _L)],
                                                 rs0[row, pl.ds(cc * _L, _L)])
                               for cc in range(nchw)]))
                sq16 = _tree([tbuf[j, pl.ds(0, _L)] for j in range(_L)])
                ee_all[pl.ds(ch * _B + g * _L, _L)] = sq16

        pltpu.sync_copy(ee_all.at[pl.ds(0, rows_per_tile)],
                        sh_sq.at[pl.ds(sqbase, rows_per_tile)])

        @pl.loop(0, accrows)
        def _zr(i):
            for j in range(128 // _L):
                acc[i, pl.ds(j * _L, _L)] = zz

        @pl.loop(0, accrows // _L)
        def _rid(i):
            rowid[pl.ds(i * _L, _L)] = i * _L + lax.iota(jnp.int32, _L)

        @pl.when(s == 0)
        def _():
            pltpu.sync_copy(acc, sh_acc)

        plsc.subcore_barrier()
        pltpu.sync_copy(sh_sq, sqtab)

        pltpu.sync_copy(src_hbm.at[pl.ds(tbase, epw)], idx_s)
        pltpu.sync_copy(dst_hbm.at[pl.ds(tbase, epw)], idx_d)
        bsc = betav[pl.ds(0, _L)]
        csc = conv[pl.ds(0, _L)]

        def fire(b, rs, rd, ss, sd):
            pltpu.async_copy(xp_hbm.at[idx_s.at[pl.ds(b * _B, _B)]], rs, ss)
            pltpu.async_copy(xp_hbm.at[idx_d.at[pl.ds(b * _B, _B)]], rd, sd)

        def wait(b, rs, rd, ss, sd):
            pltpu.make_async_copy(
                xp_hbm.at[idx_s.at[pl.ds(b * _B, _B)]], rs, ss).wait()
            pltpu.make_async_copy(
                xp_hbm.at[idx_d.at[pl.ds(b * _B, _B)]], rd, sd).wait()

        def compute(b, rs, rd):
            @pl.loop(0, ngrp)
            def _grp(g):
                off = b * _B + g * _L
                iv_s = idx_s[pl.ds(off, _L)]
                iv_d = idx_d[pl.ds(off, _L)]
                s1 = plsc.load_gather(sqtab, [iv_s])
                s2 = plsc.load_gather(sqtab, [iv_d])
                for ee_ in range(_L):
                    row = g * _L + ee_
                    plsc.store_scatter(
                        tbuf, [lane, jnp.full((_L,), ee_, jnp.int32)],
                        _tree([_bf16_pair_mulsum(rs[row, pl.ds(cc * _L, _L)],
                                                 rd[row, pl.ds(cc * _L, _L)])
                               for cc in range(nchw)]))
                dd = _tree([tbuf[j, pl.ds(0, _L)] for j in range(_L)])
                am = 1.0 - 2.0 * dd + s2
                bm = 1.0 - s1
                den = jnp.maximum(1.0 - 2.0 * dd + s1 * s2, 1e-15)
                num2 = jnp.maximum(
                    am * am * s1 - 2.0 * am * bm * dd + bm * bm * s2, 0.0)
                ma2 = num2 / (den * den)
                z = jnp.minimum(_sc_sqrt(ma2), jnp.float32(1.0 - 1e-7))
                a = 0.5 * _sc_log((1.0 + z) / (1.0 - z))
                ev = _sc_tanh(bsc * (4.0 * a * a) + csc)
                ee_all[pl.ds(off, _L)] = ev
                r = lax.shift_right_logical(iv_s, 7)
                col = jnp.bitwise_and(iv_s, 127)
                plsc.addupdate_scatter(acc, [r, col], jnp.abs(ev))

        fire(0, rs0, rd0, ss0, sd0)

        @pl.loop(0, nblk - 1, step=2)
        def _blk(bb):
            fire(bb + 1, rs1, rd1, ss1, sd1)
            wait(bb, rs0, rd0, ss0, sd0)
            compute(bb, rs0, rd0)
            fire(bb + 2, rs0, rd0, ss0, sd0)
            wait(bb + 1, rs1, rd1, ss1, sd1)
            compute(bb + 1, rs1, rd1)

        wait(nblk - 1, rs0, rd0, ss0, sd0)
        compute(nblk - 1, rs0, rd0)
        pltpu.sync_copy(ee_all, ee_hbm.at[pl.ds(tbase, epw)])

        pltpu.sync_copy(acc, sh_acc.at[rowid], add=True)
        plsc.subcore_barrier()

        @pl.when(s == 0)
        def _():
            pltpu.sync_copy(sh_acc, part_hbm.at[c])

    return k(xp, src, dst, beta, con)


def kernel(x, edge_index, beta, con):
    n, d = x.shape
    npad = ((n + (_NS * 128) - 1) // (_NS * 128)) * (_NS * 128)
    src = edge_index[0]
    dst = edge_index[1]
    xp = lax.bitcast_convert_type(
        x.astype(jnp.bfloat16).reshape(n, d // 2, 2), jnp.int32)
    xp = jnp.pad(xp, ((0, npad - n), (0, 0)))
    beta16 = jnp.broadcast_to(beta.astype(jnp.float32), (_L,))
    con16 = jnp.broadcast_to(con.astype(jnp.float32), (_L,))
    edge_e, parts = _fused(xp, src, dst, beta16, con16)
    rowsum = parts.reshape(_NC, -1).sum(axis=0)[:n] + 1e-10
    return edge_e, rowsum[:, None]
